# Initial kernel scaffold; baseline (speedup 1.0000x reference)
import functools

import jax
import jax.numpy as jnp
from jax import lax
from jax.experimental import pallas as pl
from jax.experimental.pallas import tpu as pltpu
from jax.experimental.pallas import tpu_sc as plsc

_NC = 2
_NS = 16
_NW = _NC * _NS
_L = 16


def _sc_call(n_nodes, n_edges, d_out, chunk):
    e_per_w = n_edges // _NW
    n_chunks = e_per_w // chunk
    mesh = plsc.VectorSubcoreMesh(core_axis_name="c", subcore_axis_name="s")

    @functools.partial(
        pl.kernel,
        out_type=jax.ShapeDtypeStruct((n_edges, d_out), jnp.float32),
        mesh=mesh,
        scratch_types=[
            pltpu.VMEM((n_nodes * 3,), jnp.float32),
            pltpu.VMEM((chunk,), jnp.int32),
            pltpu.VMEM((chunk,), jnp.int32),
            pltpu.VMEM((chunk, d_out), jnp.float32),
            pltpu.SemaphoreType.DMA,
        ],
    )
    def sc_kernel(table_hbm, posf_hbm, src_hbm, dst_hbm, out_hbm,
                  posv, srcv, dstv, buf, sem):
        wid = lax.axis_index("s") * _NC + lax.axis_index("c")
        pltpu.sync_copy(posf_hbm, posv)

        def chunk_body(j, carry):
            base = wid * e_per_w + j * chunk
            pltpu.sync_copy(src_hbm.at[pl.ds(base, chunk)], srcv)
            pltpu.sync_copy(dst_hbm.at[pl.ds(base, chunk)], dstv)
            pltpu.async_copy(table_hbm.at[srcv], buf, sem).wait()

            def grp(i, c2):
                rows = jnp.arange(_L, dtype=jnp.int32) + i * _L
                d16 = dstv[pl.ds(i * _L, _L)]
                for c in range(3):
                    cc = jnp.full((_L,), c, dtype=jnp.int32)
                    pd = plsc.load_gather(posv, [d16 * 3 + c])
                    cur = plsc.load_gather(buf, [rows, cc])
                    plsc.store_scatter(buf, [rows, cc], cur - pd)
                return c2

            lax.fori_loop(0, chunk // _L, grp, 0)
            pltpu.sync_copy(buf, out_hbm.at[pl.ds(base, chunk)])
            return carry

        lax.fori_loop(0, n_chunks, chunk_body, 0)

    return sc_kernel


def kernel(pos, feat, edge_index):
    n_nodes, d_feat = feat.shape
    n_edges = edge_index.shape[1]
    d_out = d_feat + 3
    table = jnp.concatenate([pos, feat], axis=1)
    pos_flat = pos.reshape(-1)
    src = edge_index[0].astype(jnp.int32)
    dst = edge_index[1].astype(jnp.int32)
    fn = _sc_call(n_nodes, n_edges, d_out, chunk=80)
    return fn(table, pos_flat, src, dst)


# speedup vs baseline: 2.2200x; 2.2200x over previous
import functools

import jax
import jax.numpy as jnp
from jax import lax
from jax.experimental import pallas as pl
from jax.experimental.pallas import tpu as pltpu
from jax.experimental.pallas import tpu_sc as plsc

_NC = 2
_NS = 16
_NW = _NC * _NS
_L = 16


def _sc_call(n_nodes, n_edges, d_feat, d_out, chunk):
    e_per_w = n_edges // _NW
    n_chunks = e_per_w // chunk
    mesh = plsc.VectorSubcoreMesh(core_axis_name="c", subcore_axis_name="s")

    @functools.partial(
        pl.kernel,
        out_type=jax.ShapeDtypeStruct((n_edges * d_out,), jnp.float32),
        mesh=mesh,
        scratch_types=[
            pltpu.VMEM((n_nodes * 3,), jnp.float32),
            pltpu.VMEM((chunk,), jnp.int32),
            pltpu.VMEM((chunk,), jnp.int32),
            pltpu.VMEM((chunk, d_feat), jnp.float32),
            pltpu.VMEM((chunk * d_out,), jnp.float32),
            pltpu.SemaphoreType.DMA,
        ],
        compiler_params=pltpu.CompilerParams(needs_layout_passes=False),
    )
    def sc_kernel(feat_hbm, posf_hbm, src_hbm, dst_hbm, out_hbm,
                  posv, srcv, dstv, fbuf, buf, sem):
        wid = lax.axis_index("s") * _NC + lax.axis_index("c")
        pltpu.sync_copy(posf_hbm, posv)
        iota = jnp.arange(_L, dtype=jnp.int32)

        def chunk_body(j, carry):
            base = wid * e_per_w + j * chunk
            pltpu.sync_copy(src_hbm.at[pl.ds(base, chunk)], srcv)
            pltpu.sync_copy(dst_hbm.at[pl.ds(base, chunk)], dstv)
            pltpu.async_copy(feat_hbm.at[srcv], fbuf, sem).wait()

            def rel_grp(i, c2):
                s16 = srcv[pl.ds(i * _L, _L)]
                d16 = dstv[pl.ds(i * _L, _L)]
                obase = (iota + i * _L) * d_out
                for c in range(3):
                    ps = plsc.load_gather(posv, [s16 * 3 + c])
                    pd = plsc.load_gather(posv, [d16 * 3 + c])
                    plsc.store_scatter(buf, [obase + c], ps - pd)
                return c2

            lax.fori_loop(0, chunk // _L, rel_grp, 0)

            def row_cp(r, c2):
                rr = jnp.full((_L,), r, dtype=jnp.int32)
                ob = r * d_out + 3 + iota
                for k in range(d_feat // _L):
                    v = plsc.load_gather(fbuf, [rr, iota + k * _L])
                    plsc.store_scatter(buf, [ob + k * _L], v)
                return c2

            lax.fori_loop(0, chunk, row_cp, 0)
            pltpu.sync_copy(buf, out_hbm.at[pl.ds(base * d_out, chunk * d_out)])
            return carry

        lax.fori_loop(0, n_chunks, chunk_body, 0)

    return sc_kernel


def kernel(pos, feat, edge_index):
    n_nodes, d_feat = feat.shape
    n_edges = edge_index.shape[1]
    d_out = d_feat + 3
    pos_flat = pos.reshape(-1)
    src = edge_index[0].astype(jnp.int32)
    dst = edge_index[1].astype(jnp.int32)
    fn = _sc_call(n_nodes, n_edges, d_feat, d_out, chunk=80)
    out_flat = fn(feat, pos_flat, src, dst)
    return out_flat.reshape(n_edges, d_out)


# speedup vs baseline: 2.7206x; 1.2255x over previous
import functools

import jax
import jax.numpy as jnp
from jax import lax
from jax.experimental import pallas as pl
from jax.experimental.pallas import tpu as pltpu
from jax.experimental.pallas import tpu_sc as plsc

_NC = 2
_NS = 16
_NW = _NC * _NS
_L = 16


def _sc_call(n_nodes, n_edges, d_feat, d_out, chunk):
    e_per_w = n_edges // _NW
    n_chunks = e_per_w // chunk
    n_iters = n_chunks + (n_chunks % 2)

    mesh = plsc.VectorSubcoreMesh(core_axis_name="c", subcore_axis_name="s")

    @functools.partial(
        pl.kernel,
        out_type=jax.ShapeDtypeStruct((n_edges * d_out,), jnp.float32),
        mesh=mesh,
        scratch_types=[
            pltpu.VMEM((n_nodes * 3,), jnp.float32),
            pltpu.VMEM((chunk,), jnp.int32),
            pltpu.VMEM((chunk,), jnp.int32),
            pltpu.VMEM((chunk,), jnp.int32),
            pltpu.VMEM((chunk,), jnp.int32),
            pltpu.VMEM((chunk, d_feat), jnp.float32),
            pltpu.VMEM((chunk, d_feat), jnp.float32),
            pltpu.VMEM((chunk * d_out,), jnp.float32),
            pltpu.VMEM((chunk * d_out,), jnp.float32),
            pltpu.SemaphoreType.DMA,
            pltpu.SemaphoreType.DMA,
            pltpu.SemaphoreType.DMA,
            pltpu.SemaphoreType.DMA,
            pltpu.SemaphoreType.DMA,
            pltpu.SemaphoreType.DMA,
            pltpu.SemaphoreType.DMA,
            pltpu.SemaphoreType.DMA,
        ],
        compiler_params=pltpu.CompilerParams(needs_layout_passes=False),
    )
    def sc_kernel(feat_hbm, posf_hbm, src_hbm, dst_hbm, out_hbm,
                  posv, sv0, sv1, dv0, dv1, fb0, fb1, bf0, bf1,
                  ssem0, ssem1, dsem0, dsem1, gsem0, gsem1, osem0, osem1):
        wid = lax.axis_index("s") * _NC + lax.axis_index("c")
        w0 = wid * e_per_w
        iota = jnp.arange(_L, dtype=jnp.int32)

        sv = (sv0, sv1)
        dv = (dv0, dv1)
        fb = (fb0, fb1)
        bf = (bf0, bf1)
        ssem = (ssem0, ssem1)
        dsem = (dsem0, dsem1)
        gsem = (gsem0, gsem1)
        osem = (osem0, osem1)

        def cbase(g):
            return w0 + jnp.minimum(g, n_chunks - 1) * chunk

        def issue_idx(g, b):
            base = cbase(g)
            pltpu.async_copy(src_hbm.at[pl.ds(base, chunk)], sv[b], ssem[b])
            pltpu.async_copy(dst_hbm.at[pl.ds(base, chunk)], dv[b], dsem[b])

        def wait_idx(b):
            pltpu.make_async_copy(src_hbm.at[pl.ds(w0, chunk)], sv[b], ssem[b]).wait()
            pltpu.make_async_copy(dst_hbm.at[pl.ds(w0, chunk)], dv[b], dsem[b]).wait()

        def issue_gather(b):
            pltpu.async_copy(feat_hbm.at[sv[b]], fb[b], gsem[b])

        def wait_gather(b):
            pltpu.make_async_copy(feat_hbm.at[sv[b]], fb[b], gsem[b]).wait()

        def issue_write(g, b):
            pltpu.async_copy(
                bf[b], out_hbm.at[pl.ds(cbase(g) * d_out, chunk * d_out)], osem[b])

        def wait_write(b):
            pltpu.make_async_copy(
                bf[b], out_hbm.at[pl.ds(w0 * d_out, chunk * d_out)], osem[b]).wait()

        def compute(b):
            svb, dvb, fbb, bfb = sv[b], dv[b], fb[b], bf[b]

            def rel_grp(i, c2):
                s16 = svb[pl.ds(i * _L, _L)]
                d16 = dvb[pl.ds(i * _L, _L)]
                obase = (iota + i * _L) * d_out
                for c in range(3):
                    ps = plsc.load_gather(posv, [s16 * 3 + c])
                    pd = plsc.load_gather(posv, [d16 * 3 + c])
                    plsc.store_scatter(bfb, [obase + c], ps - pd)
                return c2

            lax.fori_loop(0, chunk // _L, rel_grp, 0)

            def row_cp(r, c2):
                rr = jnp.full((_L,), r, dtype=jnp.int32)
                ob = r * d_out + 3 + iota
                for k in range(d_feat // _L):
                    v = plsc.load_gather(fbb, [rr, iota + k * _L])
                    plsc.store_scatter(bfb, [ob + k * _L], v)
                return c2

            lax.fori_loop(0, chunk, row_cp, 0)

        def do_iter(g, b, i):
            wait_gather(b)
            wait_idx(1 - b)
            issue_gather(1 - b)

            @pl.when(i >= 1)
            def _():
                wait_write(b)

            compute(b)
            issue_write(g, b)
            issue_idx(g + 2, b)

        pltpu.sync_copy(posf_hbm, posv)
        issue_idx(0, 0)
        issue_idx(1, 1)
        wait_idx(0)
        issue_gather(0)

        def pair(i, carry):
            g0 = 2 * i
            do_iter(g0, 0, i)
            do_iter(g0 + 1, 1, i)
            return carry

        lax.fori_loop(0, n_iters // 2, pair, 0)

        wait_gather(0)
        wait_idx(1)
        wait_write(0)
        wait_write(1)

    return sc_kernel


def kernel(pos, feat, edge_index):
    n_nodes, d_feat = feat.shape
    n_edges = edge_index.shape[1]
    d_out = d_feat + 3
    pos_flat = pos.reshape(-1)
    src = edge_index[0].astype(jnp.int32)
    dst = edge_index[1].astype(jnp.int32)
    fn = _sc_call(n_nodes, n_edges, d_feat, d_out, chunk=80)
    out_flat = fn(feat, pos_flat, src, dst)
    return out_flat.reshape(n_edges, d_out)


# speedup vs baseline: 4.0578x; 1.4915x over previous
import functools

import jax
import jax.numpy as jnp
from jax import lax
from jax.experimental import pallas as pl
from jax.experimental.pallas import tpu as pltpu
from jax.experimental.pallas import tpu_sc as plsc

_NC = 2
_NS = 16
_NW = _NC * _NS
_L = 16


def _sc_call(n_nodes, n_edges, d_feat, d_out, chunk):
    e_per_w = n_edges // _NW
    n_chunks = e_per_w // chunk
    n_iters = n_chunks + (n_chunks % 2)

    mesh = plsc.VectorSubcoreMesh(core_axis_name="c", subcore_axis_name="s")

    @functools.partial(
        pl.kernel,
        out_type=jax.ShapeDtypeStruct((n_edges, d_out), jnp.float32),
        mesh=mesh,
        scratch_types=[
            pltpu.VMEM((n_nodes * 3,), jnp.float32),
            pltpu.VMEM((chunk,), jnp.int32),
            pltpu.VMEM((chunk,), jnp.int32),
            pltpu.VMEM((chunk,), jnp.int32),
            pltpu.VMEM((chunk,), jnp.int32),
            pltpu.VMEM((chunk, d_feat), jnp.float32),
            pltpu.VMEM((chunk, d_feat), jnp.float32),
            pltpu.VMEM((chunk, d_out), jnp.float32),
            pltpu.VMEM((chunk, d_out), jnp.float32),
            pltpu.SemaphoreType.DMA,
            pltpu.SemaphoreType.DMA,
            pltpu.SemaphoreType.DMA,
            pltpu.SemaphoreType.DMA,
            pltpu.SemaphoreType.DMA,
            pltpu.SemaphoreType.DMA,
            pltpu.SemaphoreType.DMA,
            pltpu.SemaphoreType.DMA,
        ],
        compiler_params=pltpu.CompilerParams(needs_layout_passes=False),
    )
    def sc_kernel(feat_hbm, posf_hbm, src_hbm, dst_hbm, out_hbm,
                  posv, sv0, sv1, dv0, dv1, fb0, fb1, bf0, bf1,
                  ssem0, ssem1, dsem0, dsem1, gsem0, gsem1, osem0, osem1):
        wid = lax.axis_index("s") * _NC + lax.axis_index("c")
        w0 = wid * e_per_w
        iota = jnp.arange(_L, dtype=jnp.int32)

        sv = (sv0, sv1)
        dv = (dv0, dv1)
        fb = (fb0, fb1)
        bf = (bf0, bf1)
        ssem = (ssem0, ssem1)
        dsem = (dsem0, dsem1)
        gsem = (gsem0, gsem1)
        osem = (osem0, osem1)

        def cbase(g):
            return w0 + jnp.minimum(g, n_chunks - 1) * chunk

        def issue_idx(g, b):
            base = cbase(g)
            pltpu.async_copy(src_hbm.at[pl.ds(base, chunk)], sv[b], ssem[b])
            pltpu.async_copy(dst_hbm.at[pl.ds(base, chunk)], dv[b], dsem[b])

        def wait_idx(b):
            pltpu.make_async_copy(src_hbm.at[pl.ds(w0, chunk)], sv[b], ssem[b]).wait()
            pltpu.make_async_copy(dst_hbm.at[pl.ds(w0, chunk)], dv[b], dsem[b]).wait()

        def issue_gather(b):
            pltpu.async_copy(feat_hbm.at[sv[b]], fb[b], gsem[b])

        def wait_gather(b):
            pltpu.make_async_copy(feat_hbm.at[sv[b]], fb[b], gsem[b]).wait()

        def issue_write(g, b):
            pltpu.async_copy(
                bf[b], out_hbm.at[pl.ds(cbase(g), chunk)], osem[b])

        def wait_write(b):
            pltpu.make_async_copy(
                bf[b], out_hbm.at[pl.ds(w0, chunk)], osem[b]).wait()

        def compute(b):
            svb, dvb, fbb, bfb = sv[b], dv[b], fb[b], bf[b]

            def rel_grp(i, c2):
                s16 = svb[pl.ds(i * _L, _L)]
                d16 = dvb[pl.ds(i * _L, _L)]
                r16 = iota + i * _L
                for c in range(3):
                    cc = jnp.full((_L,), c, dtype=jnp.int32)
                    ps = plsc.load_gather(posv, [s16 * 3 + c])
                    pd = plsc.load_gather(posv, [d16 * 3 + c])
                    plsc.store_scatter(bfb, [r16, cc], ps - pd)
                return c2

            lax.fori_loop(0, chunk // _L, rel_grp, 0)

            def row_cp(r, c2):
                rr = jnp.full((_L,), r, dtype=jnp.int32)
                for k in range(d_feat // _L):
                    v = plsc.load_gather(fbb, [rr, iota + k * _L])
                    plsc.store_scatter(bfb, [rr, iota + 3 + k * _L], v)
                return c2

            lax.fori_loop(0, chunk, row_cp, 0)

        def do_iter(g, b, i):
            wait_gather(b)
            wait_idx(1 - b)
            issue_gather(1 - b)

            @pl.when(i >= 1)
            def _():
                wait_write(b)

            compute(b)
            issue_write(g, b)
            issue_idx(g + 2, b)

        pltpu.sync_copy(posf_hbm, posv)
        issue_idx(0, 0)
        issue_idx(1, 1)
        wait_idx(0)
        issue_gather(0)

        def pair(i, carry):
            g0 = 2 * i
            do_iter(g0, 0, i)
            do_iter(g0 + 1, 1, i)
            return carry

        lax.fori_loop(0, n_iters // 2, pair, 0)

        wait_gather(0)
        wait_idx(1)
        wait_write(0)
        wait_write(1)

    return sc_kernel


def kernel(pos, feat, edge_index):
    n_nodes, d_feat = feat.shape
    n_edges = edge_index.shape[1]
    d_out = d_feat + 3
    pos_flat = pos.reshape(-1)
    src = edge_index[0].astype(jnp.int32)
    dst = edge_index[1].astype(jnp.int32)
    fn = _sc_call(n_nodes, n_edges, d_feat, d_out, chunk=80)
    return fn(feat, pos_flat, src, dst)
